# baseline (speedup 1.0000x reference)
import functools

import jax
import jax.numpy as jnp
import numpy as np
from jax import lax
from jax.experimental import pallas as pl
from jax.experimental.pallas import tpu as pltpu
from jax.experimental.pallas import tpu_sc as plsc

_NEG = np.float32(-1e9)
_BLK = 512
_NT = (((1,), (1,)), ((), ()))


def _sc_gather_masks(tA, t1, tL, t2, ati, o1i, loci, o2i, O, L):
    B = ati.shape[0]
    info = plsc.get_sparse_core_info()
    nw = info.num_cores * info.num_subcores
    bw = B // nw
    mesh = plsc.VectorSubcoreMesh(core_axis_name="c", subcore_axis_name="s")
    f32 = jnp.float32

    @functools.partial(
        pl.kernel,
        mesh=mesh,
        out_type=tuple(jax.ShapeDtypeStruct((B,), f32) for _ in range(4)),
        scratch_types=[
            pltpu.VMEM((4, bw), jnp.int32),
            pltpu.VMEM((4, bw), jnp.int32),
            pltpu.VMEM((bw, 128), f32),
            pltpu.VMEM((bw, 128), f32),
            pltpu.VMEM((bw, 128), f32),
            pltpu.VMEM((bw, 128), f32),
            pltpu.VMEM((bw,), f32),
            pltpu.SemaphoreType.DMA,
        ],
    )
    def k(tA_h, t1_h, tL_h, t2_h, at_h, o1_h, loc_h, o2_h,
          oA_h, o1m_h, oL_h, o2m_h, iv, idx_v, bA_v, b1_v, bL_v, b2_v,
          out_v, sem):
        wid = lax.axis_index("s") * info.num_cores + lax.axis_index("c")
        base = wid * bw
        cb = (base // 128) * 128
        co = base - cb
        for i, h in enumerate((at_h, o1_h, loc_h, o2_h)):
            pltpu.sync_copy(h.at[pl.ds(base, bw)], iv.at[i])
        for g in range(bw // 16):
            sl = pl.ds(g * 16, 16)
            atv = iv[0, sl]
            o1v = iv[1, sl]
            locv = iv[2, sl]
            o2v = iv[3, sl]
            idx_v[0, sl] = atv
            idx_v[1, sl] = atv * O + o1v
            idx_v[2, sl] = atv * L + locv
            idx_v[3, sl] = (atv * O + o1v) * O + o2v
        copies = [
            pltpu.async_copy(t_h.at[idx_v.at[i], pl.ds(cb, 128)], b_v, sem)
            for i, (t_h, b_v) in enumerate(
                ((tA_h, bA_v), (t1_h, b1_v), (tL_h, bL_v), (t2_h, b2_v)))
        ]
        for c in copies:
            c.wait()
        it = lax.iota(jnp.int32, 16)
        for o_h, b_v in ((oA_h, bA_v), (o1m_h, b1_v), (oL_h, bL_v),
                         (o2m_h, b2_v)):
            for g in range(bw // 16):
                acc = jnp.zeros((16,), f32)
                for r in range(16):
                    row = b_v[g * 16 + r, pl.ds(co + g * 16, 16)]
                    acc = jnp.where(it == r, row, acc)
                out_v[pl.ds(g * 16, 16)] = acc
            pltpu.sync_copy(out_v, o_h.at[pl.ds(base, bw)])

    return k(tA, t1, tL, t2, ati, o1i, loci, o2i)


def _scores_body(enc_ref, watt_ref, w1t_ref, wlt_ref, w2t_ref,
                 at_ref, o1_ref, loc_ref, o2_ref,
                 bat_ref, bo1_ref, bloc_ref, bo2_ref,
                 oA_ref, o1_ref_out, oL_ref, o2_ref_out):
    f32 = jnp.float32
    H = enc_ref.shape[1]
    A = watt_ref.shape[0]
    O = w1t_ref.shape[0]
    n = enc_ref.shape[0]
    enc = enc_ref[...]

    def nt(x, y):
        return lax.dot_general(x, y, _NT, preferred_element_type=f32)

    def to_col(ref):
        t = jnp.swapaxes(ref[0], 0, 1)
        return jnp.concatenate(
            [t[:, i:i + 1] for i in range(t.shape[1])], axis=0)

    at = to_col(at_ref)
    o1 = to_col(o1_ref)
    loc = to_col(loc_ref)
    o2 = to_col(o2_ref)
    ioA = lax.broadcasted_iota(jnp.int32, (n, A), 1)
    io64 = lax.broadcasted_iota(jnp.int32, (n, O), 1)
    bf16 = jnp.bfloat16
    oh_at = (ioA == at).astype(bf16)
    oh_o1 = (io64 == o1).astype(bf16)
    oh_loc = (io64 == loc).astype(bf16)
    oh_o2 = (io64 == o2).astype(bf16)
    oh_atf = oh_at.astype(f32)
    oh_o1f = oh_o1.astype(f32)
    oh_locf = oh_loc.astype(f32)
    oh_o2f = oh_o2.astype(f32)

    w1t = w1t_ref[...]
    wlt = wlt_ref[...]
    w2t = w2t_ref[...]

    row_at = nt(enc, watt_ref[...]) + bat_ref[...]
    oA_ref[...] = jnp.sum(oh_atf * row_at, axis=1)

    row1 = nt(enc, w1t[:, :H]) + nt(oh_at, w1t[:, H:]) + bo1_ref[...]
    o1_ref_out[...] = jnp.sum(oh_o1f * row1, axis=1)

    rowL = (nt(enc, wlt[:, :H]) + nt(oh_at, wlt[:, H:H + A])
            + nt(oh_o1, wlt[:, H + A:]) + bloc_ref[...])
    oL_ref[...] = jnp.sum(oh_locf * rowL, axis=1)

    row2 = (nt(enc, w2t[:, :H]) + nt(oh_at, w2t[:, H:H + A])
            + nt(oh_o1, w2t[:, H + A:H + A + O])
            + nt(oh_loc, w2t[:, H + A + O:]) + bo2_ref[...])
    o2_ref_out[...] = jnp.sum(oh_o2f * row2, axis=1)


def _tc_scores(enc, watt, w1t, wlt, w2t, at, o1, loc, o2,
               b_at, b_o1, b_loc, b_o2):
    B, H = enc.shape

    def rows(i):
        return (i, 0)

    def full(i):
        return (0, 0)

    def fixed(a):
        return pl.BlockSpec(a.shape, full)

    def mrow(a):
        return pl.BlockSpec((1, _BLK // 128, 128), lambda i: (i, 0, 0))

    ospec = pl.BlockSpec((_BLK,), lambda i: (i,))
    oshape = jax.ShapeDtypeStruct((B,), jnp.float32)
    return pl.pallas_call(
        _scores_body,
        grid=(B // _BLK,),
        in_specs=[
            pl.BlockSpec((_BLK, H), rows),
            fixed(watt), fixed(w1t), fixed(wlt), fixed(w2t),
            mrow(at), mrow(o1), mrow(loc), mrow(o2),
            fixed(b_at), fixed(b_o1), fixed(b_loc), fixed(b_o2),
        ],
        out_specs=(ospec, ospec, ospec, ospec),
        out_shape=(oshape, oshape, oshape, oshape),
    )(enc, watt, w1t, wlt, w2t, at, o1, loc, o2, b_at, b_o1, b_loc, b_o2)


def _combine_body(sA_ref, s1_ref, sL_ref, s2_ref,
                  mA_ref, m1_ref, mL_ref, m2_ref, out_ref):
    q = jnp.where(mA_ref[...] > 0.0, sA_ref[...], _NEG)
    q = q + jnp.where(m1_ref[...] > 0.0, s1_ref[...], _NEG)
    q = q + jnp.where(mL_ref[...] > 0.0, sL_ref[...], _NEG)
    q = q + jnp.where(m2_ref[...] > 0.0, s2_ref[...], _NEG)
    out_ref[...] = q


def _tc_combine(sA, s1, sL, s2, mA, m1, mL, m2):
    spec = pl.BlockSpec(sA.shape, lambda: (0, 0))
    return pl.pallas_call(
        _combine_body,
        in_specs=[spec] * 8,
        out_specs=spec,
        out_shape=jax.ShapeDtypeStruct(sA.shape, jnp.float32),
    )(sA, s1, sL, s2, mA, m1, mL, m2)


def kernel(encoded, action_types, object1, location, object2,
           action_type_masks, object1_masks, location_masks, object2_masks,
           W_at, b_at, W_o1, b_o1, W_loc, b_loc, W_o2, b_o2):
    B, H = encoded.shape
    A = action_type_masks.shape[1]
    O = object1_masks.shape[2]
    L = location_masks.shape[2]

    at = action_types[:, 0].astype(jnp.int32)
    o1i = object1[:, 0].astype(jnp.int32)
    loci = location[:, 0].astype(jnp.int32)
    o2i = object2[:, 0].astype(jnp.int32)

    tA = action_type_masks.transpose(1, 0)
    t1 = object1_masks.transpose(1, 2, 0).reshape(A * O, B)
    tL = location_masks.transpose(1, 2, 0).reshape(A * L, B)
    t2 = object2_masks.transpose(1, 2, 3, 0).reshape(A * O * O, B)

    mA, m1, mL, m2 = _sc_gather_masks(tA, t1, tL, t2, at, o1i, loci, o2i,
                                      O, L)

    i3 = (B // _BLK, _BLK // 128, 128)
    bf16 = jnp.bfloat16
    sA, s1, sL, s2 = _tc_scores(
        encoded.astype(bf16),
        W_at.transpose(1, 0).astype(bf16), W_o1.transpose(1, 0).astype(bf16),
        W_loc.transpose(1, 0).astype(bf16),
        W_o2.transpose(1, 0).astype(bf16),
        at.reshape(i3), o1i.reshape(i3), loci.reshape(i3), o2i.reshape(i3),
        b_at.reshape(1, A), b_o1.reshape(1, O), b_loc.reshape(1, L),
        b_o2.reshape(1, O))

    t8 = (B // 128, 128)
    q = _tc_combine(sA.reshape(t8), s1.reshape(t8), sL.reshape(t8),
                    s2.reshape(t8), mA.reshape(t8), m1.reshape(t8),
                    mL.reshape(t8), m2.reshape(t8))
    return q.reshape(B, 1)


# speedup vs baseline: 1.1965x; 1.1965x over previous
import functools

import jax
import jax.numpy as jnp
import numpy as np
from jax import lax
from jax.experimental import pallas as pl
from jax.experimental.pallas import tpu as pltpu
from jax.experimental.pallas import tpu_sc as plsc

_NEG = np.float32(-1e9)
_BLK = 512
_NT = (((1,), (1,)), ((), ()))


def _sc_gather_masks(tA, t1, tL, t2, ati, o1i, loci, o2i, O, L):
    B = ati.shape[0]
    info = plsc.get_sparse_core_info()
    nw = info.num_cores * info.num_subcores
    bw = B // nw
    mesh = plsc.VectorSubcoreMesh(core_axis_name="c", subcore_axis_name="s")
    f32 = jnp.float32

    @functools.partial(
        pl.kernel,
        mesh=mesh,
        out_type=tuple(jax.ShapeDtypeStruct((B,), f32) for _ in range(4)),
        scratch_types=[
            pltpu.VMEM((4, bw), jnp.int32),
            pltpu.VMEM((4, bw), jnp.int32),
            pltpu.VMEM((bw,), f32),
            pltpu.SemaphoreType.DMA,
        ],
    )
    def k(tA_h, t1_h, tL_h, t2_h, at_h, o1_h, loc_h, o2_h,
          oA_h, o1m_h, oL_h, o2m_h, iv, idx_v, out_v, sem):
        wid = lax.axis_index("s") * info.num_cores + lax.axis_index("c")
        base = wid * bw
        cb = (base // 128) * 128
        co = base - cb
        for i, h in enumerate((at_h, o1_h, loc_h, o2_h)):
            pltpu.sync_copy(h.at[pl.ds(base, bw)], iv.at[i])
        for g in range(bw // 16):
            sl = pl.ds(g * 16, 16)
            atv = iv[0, sl]
            o1v = iv[1, sl]
            locv = iv[2, sl]
            o2v = iv[3, sl]
            idx_v[0, sl] = atv
            idx_v[1, sl] = atv * O + o1v
            idx_v[2, sl] = atv * L + locv
            idx_v[3, sl] = (atv * O + o1v) * O + o2v

        def body(bA_v, b1_v, bL_v, b2_v):
            copies = [
                pltpu.async_copy(t_h.at[idx_v.at[i], pl.ds(cb, 128)], b_v,
                                 sem)
                for i, (t_h, b_v) in enumerate(
                    ((tA_h, bA_v), (t1_h, b1_v), (tL_h, bL_v), (t2_h, b2_v)))
            ]
            for c in copies:
                c.wait()
            it = lax.iota(jnp.int32, 16)
            for o_h, b_v in ((oA_h, bA_v), (o1m_h, b1_v), (oL_h, bL_v),
                             (o2m_h, b2_v)):
                for g in range(bw // 16):
                    acc = jnp.zeros((16,), f32)
                    for r in range(16):
                        row = b_v[g * 16 + r, pl.ds(co + g * 16, 16)]
                        acc = jnp.where(it == r, row, acc)
                    out_v[pl.ds(g * 16, 16)] = acc
                pltpu.sync_copy(out_v, o_h.at[pl.ds(base, bw)])

        pl.run_scoped(body, *[pltpu.VMEM((bw, 128), f32)] * 4)

    return k(tA, t1, tL, t2, ati, o1i, loci, o2i)


def _scores_body(enc_ref, watt_ref, w1t_ref, wlt_ref, w2t_ref,
                 at_ref, o1_ref, loc_ref, o2_ref,
                 bat_ref, bo1_ref, bloc_ref, bo2_ref,
                 oA_ref, o1_ref_out, oL_ref, o2_ref_out):
    f32 = jnp.float32
    H = enc_ref.shape[1]
    A = watt_ref.shape[0]
    O = w1t_ref.shape[0]
    n = enc_ref.shape[0]
    enc = enc_ref[...]

    def nt(x, y):
        return lax.dot_general(x, y, _NT, preferred_element_type=f32)

    def to_col(ref):
        t = jnp.swapaxes(ref[0], 0, 1)
        return jnp.concatenate(
            [t[:, i:i + 1] for i in range(t.shape[1])], axis=0)

    at = to_col(at_ref)
    o1 = to_col(o1_ref)
    loc = to_col(loc_ref)
    o2 = to_col(o2_ref)
    ioA = lax.broadcasted_iota(jnp.int32, (n, A), 1)
    io64 = lax.broadcasted_iota(jnp.int32, (n, O), 1)
    oh_at = (ioA == at).astype(f32)
    oh_o1 = (io64 == o1).astype(f32)
    oh_loc = (io64 == loc).astype(f32)
    oh_o2 = (io64 == o2).astype(f32)

    w1t = w1t_ref[...]
    wlt = wlt_ref[...]
    w2t = w2t_ref[...]

    row_at = nt(enc, watt_ref[...]) + bat_ref[...]
    oA_ref[...] = jnp.sum(oh_at * row_at, axis=1)

    row1 = nt(enc, w1t[:, :H]) + nt(oh_at, w1t[:, H:]) + bo1_ref[...]
    o1_ref_out[...] = jnp.sum(oh_o1 * row1, axis=1)

    rowL = (nt(enc, wlt[:, :H]) + nt(oh_at, wlt[:, H:H + A])
            + nt(oh_o1, wlt[:, H + A:]) + bloc_ref[...])
    oL_ref[...] = jnp.sum(oh_loc * rowL, axis=1)

    row2 = (nt(enc, w2t[:, :H]) + nt(oh_at, w2t[:, H:H + A])
            + nt(oh_o1, w2t[:, H + A:H + A + O])
            + nt(oh_loc, w2t[:, H + A + O:]) + bo2_ref[...])
    o2_ref_out[...] = jnp.sum(oh_o2 * row2, axis=1)


def _tc_scores(enc, watt, w1t, wlt, w2t, at, o1, loc, o2,
               b_at, b_o1, b_loc, b_o2):
    B, H = enc.shape

    def rows(i):
        return (i, 0)

    def full(i):
        return (0, 0)

    def fixed(a):
        return pl.BlockSpec(a.shape, full)

    def mrow(a):
        return pl.BlockSpec((1, _BLK // 128, 128), lambda i: (i, 0, 0))

    ospec = pl.BlockSpec((_BLK,), lambda i: (i,))
    oshape = jax.ShapeDtypeStruct((B,), jnp.float32)
    return pl.pallas_call(
        _scores_body,
        grid=(B // _BLK,),
        in_specs=[
            pl.BlockSpec((_BLK, H), rows),
            fixed(watt), fixed(w1t), fixed(wlt), fixed(w2t),
            mrow(at), mrow(o1), mrow(loc), mrow(o2),
            fixed(b_at), fixed(b_o1), fixed(b_loc), fixed(b_o2),
        ],
        out_specs=(ospec, ospec, ospec, ospec),
        out_shape=(oshape, oshape, oshape, oshape),
    )(enc, watt, w1t, wlt, w2t, at, o1, loc, o2, b_at, b_o1, b_loc, b_o2)


def _combine_body(sA_ref, s1_ref, sL_ref, s2_ref,
                  mA_ref, m1_ref, mL_ref, m2_ref, out_ref):
    q = jnp.where(mA_ref[...] > 0.0, sA_ref[...], _NEG)
    q = q + jnp.where(m1_ref[...] > 0.0, s1_ref[...], _NEG)
    q = q + jnp.where(mL_ref[...] > 0.0, sL_ref[...], _NEG)
    q = q + jnp.where(m2_ref[...] > 0.0, s2_ref[...], _NEG)
    out_ref[...] = q


def _tc_combine(sA, s1, sL, s2, mA, m1, mL, m2):
    spec = pl.BlockSpec(sA.shape, lambda: (0, 0))
    return pl.pallas_call(
        _combine_body,
        in_specs=[spec] * 8,
        out_specs=spec,
        out_shape=jax.ShapeDtypeStruct(sA.shape, jnp.float32),
    )(sA, s1, sL, s2, mA, m1, mL, m2)


def kernel(encoded, action_types, object1, location, object2,
           action_type_masks, object1_masks, location_masks, object2_masks,
           W_at, b_at, W_o1, b_o1, W_loc, b_loc, W_o2, b_o2):
    B, H = encoded.shape
    A = action_type_masks.shape[1]
    O = object1_masks.shape[2]
    L = location_masks.shape[2]

    at = action_types[:, 0].astype(jnp.int32)
    o1i = object1[:, 0].astype(jnp.int32)
    loci = location[:, 0].astype(jnp.int32)
    o2i = object2[:, 0].astype(jnp.int32)

    tA = action_type_masks.transpose(1, 0)
    t1 = object1_masks.transpose(1, 2, 0).reshape(A * O, B)
    tL = location_masks.transpose(1, 2, 0).reshape(A * L, B)
    t2 = object2_masks.transpose(1, 2, 3, 0).reshape(A * O * O, B)

    mA, m1, mL, m2 = _sc_gather_masks(tA, t1, tL, t2, at, o1i, loci, o2i,
                                      O, L)

    i3 = (B // _BLK, _BLK // 128, 128)
    sA, s1, sL, s2 = _tc_scores(
        encoded,
        W_at.transpose(1, 0), W_o1.transpose(1, 0), W_loc.transpose(1, 0),
        W_o2.transpose(1, 0),
        at.reshape(i3), o1i.reshape(i3), loci.reshape(i3), o2i.reshape(i3),
        b_at.reshape(1, A), b_o1.reshape(1, O), b_loc.reshape(1, L),
        b_o2.reshape(1, O))

    t8 = (B // 128, 128)
    q = _tc_combine(sA.reshape(t8), s1.reshape(t8), sL.reshape(t8),
                    s2.reshape(t8), mA.reshape(t8), m1.reshape(t8),
                    mL.reshape(t8), m2.reshape(t8))
    return q.reshape(B, 1)


# speedup vs baseline: 1.2116x; 1.0126x over previous
import functools

import jax
import jax.numpy as jnp
import numpy as np
from jax import lax
from jax.experimental import pallas as pl
from jax.experimental.pallas import tpu as pltpu
from jax.experimental.pallas import tpu_sc as plsc

_NEG = np.float32(-1e9)
_BLK = 512
_NT = (((1,), (1,)), ((), ()))


def _sc_gather_masks(tA, t1, tL, t2, jstk, O, L):
    B = jstk.shape[1]
    info = plsc.get_sparse_core_info()
    nw = info.num_cores * info.num_subcores
    bw = B // nw
    mesh = plsc.VectorSubcoreMesh(core_axis_name="c", subcore_axis_name="s")
    f32 = jnp.float32

    @functools.partial(
        pl.kernel,
        mesh=mesh,
        out_type=tuple(jax.ShapeDtypeStruct((B,), f32) for _ in range(4)),
        scratch_types=[
            pltpu.VMEM((4, bw), jnp.int32),
            pltpu.VMEM((4, bw), jnp.int32),
            pltpu.VMEM((4, bw), f32),
            pltpu.SemaphoreType.DMA,
            pltpu.SemaphoreType.DMA,
            pltpu.SemaphoreType.DMA,
            pltpu.SemaphoreType.DMA,
            pltpu.SemaphoreType.DMA,
        ],
    )
    def k(tA_h, t1_h, tL_h, t2_h, j_h,
          oA_h, o1m_h, oL_h, o2m_h, iv, idx_v, outb,
          s0, s1, s2, s3, so):
        wid = lax.axis_index("s") * info.num_cores + lax.axis_index("c")
        base = wid * bw
        cb = (base // 128) * 128
        co = base - cb
        for i in range(4):
            pltpu.sync_copy(j_h.at[i, pl.ds(base, bw)], iv.at[i])
        for g in range(bw // 16):
            sl = pl.ds(g * 16, 16)
            atv = iv[0, sl]
            o1v = iv[1, sl]
            locv = iv[2, sl]
            o2v = iv[3, sl]
            idx_v[0, sl] = atv
            idx_v[1, sl] = atv * O + o1v
            idx_v[2, sl] = atv * L + locv
            idx_v[3, sl] = (atv * O + o1v) * O + o2v

        def body(bA_v, b1_v, bL_v, b2_v):
            tabs = ((tA_h, bA_v, s0, oA_h), (t1_h, b1_v, s1, o1m_h),
                    (tL_h, bL_v, s2, oL_h), (t2_h, b2_v, s3, o2m_h))
            copies = [
                pltpu.async_copy(t_h.at[idx_v.at[i], pl.ds(cb, 128)], b_v,
                                 sem)
                for i, (t_h, b_v, sem, _) in enumerate(tabs)
            ]
            it = lax.iota(jnp.int32, 16)
            outs = []
            for i, (_, b_v, _, o_h) in enumerate(tabs):
                copies[i].wait()
                for g in range(bw // 16):
                    acc = jnp.zeros((16,), f32)
                    for r in range(16):
                        row = b_v[g * 16 + r, pl.ds(co + g * 16, 16)]
                        acc = jnp.where(it == r, row, acc)
                    outb[i, pl.ds(g * 16, 16)] = acc
                outs.append(
                    pltpu.async_copy(outb.at[i], o_h.at[pl.ds(base, bw)],
                                     so))
            for o in outs:
                o.wait()

        pl.run_scoped(body, *[pltpu.VMEM((bw, 128), f32)] * 4)

    return k(tA, t1, tL, t2, jstk)


def _scores_body(enc_ref, watt_ref, w1t_ref, wlt_ref, w2t_ref,
                 at_ref, o1_ref, loc_ref, o2_ref,
                 bat_ref, bo1_ref, bloc_ref, bo2_ref,
                 oA_ref, o1_ref_out, oL_ref, o2_ref_out):
    f32 = jnp.float32
    H = enc_ref.shape[1]
    A = watt_ref.shape[0]
    O = w1t_ref.shape[0]
    n = enc_ref.shape[0]
    enc = enc_ref[...]

    def nt(x, y):
        return lax.dot_general(x, y, _NT, preferred_element_type=f32)

    def to_col(ref):
        t = jnp.swapaxes(ref[0], 0, 1)
        return jnp.concatenate(
            [t[:, i:i + 1] for i in range(t.shape[1])], axis=0)

    at = to_col(at_ref)
    o1 = to_col(o1_ref)
    loc = to_col(loc_ref)
    o2 = to_col(o2_ref)
    ioA = lax.broadcasted_iota(jnp.int32, (n, A), 1)
    io64 = lax.broadcasted_iota(jnp.int32, (n, O), 1)
    oh_at = (ioA == at).astype(f32)
    oh_o1 = (io64 == o1).astype(f32)
    oh_loc = (io64 == loc).astype(f32)
    oh_o2 = (io64 == o2).astype(f32)

    w1t = w1t_ref[...]
    wlt = wlt_ref[...]
    w2t = w2t_ref[...]

    row_at = nt(enc, watt_ref[...]) + bat_ref[...]
    oA_ref[...] = jnp.sum(oh_at * row_at, axis=1)

    row1 = nt(enc, w1t[:, :H]) + nt(oh_at, w1t[:, H:]) + bo1_ref[...]
    o1_ref_out[...] = jnp.sum(oh_o1 * row1, axis=1)

    rowL = (nt(enc, wlt[:, :H]) + nt(oh_at, wlt[:, H:H + A])
            + nt(oh_o1, wlt[:, H + A:]) + bloc_ref[...])
    oL_ref[...] = jnp.sum(oh_loc * rowL, axis=1)

    row2 = (nt(enc, w2t[:, :H]) + nt(oh_at, w2t[:, H:H + A])
            + nt(oh_o1, w2t[:, H + A:H + A + O])
            + nt(oh_loc, w2t[:, H + A + O:]) + bo2_ref[...])
    o2_ref_out[...] = jnp.sum(oh_o2 * row2, axis=1)


def _tc_scores(enc, watt, w1t, wlt, w2t, at, o1, loc, o2,
               b_at, b_o1, b_loc, b_o2):
    B, H = enc.shape

    def rows(i):
        return (i, 0)

    def full(i):
        return (0, 0)

    def fixed(a):
        return pl.BlockSpec(a.shape, full)

    def mrow(a):
        return pl.BlockSpec((1, _BLK // 128, 128), lambda i: (i, 0, 0))

    ospec = pl.BlockSpec((_BLK,), lambda i: (i,))
    oshape = jax.ShapeDtypeStruct((B,), jnp.float32)
    return pl.pallas_call(
        _scores_body,
        grid=(B // _BLK,),
        in_specs=[
            pl.BlockSpec((_BLK, H), rows),
            fixed(watt), fixed(w1t), fixed(wlt), fixed(w2t),
            mrow(at), mrow(o1), mrow(loc), mrow(o2),
            fixed(b_at), fixed(b_o1), fixed(b_loc), fixed(b_o2),
        ],
        out_specs=(ospec, ospec, ospec, ospec),
        out_shape=(oshape, oshape, oshape, oshape),
    )(enc, watt, w1t, wlt, w2t, at, o1, loc, o2, b_at, b_o1, b_loc, b_o2)


def _combine_body(sA_ref, s1_ref, sL_ref, s2_ref,
                  mA_ref, m1_ref, mL_ref, m2_ref, out_ref):
    q = jnp.where(mA_ref[...] > 0.0, sA_ref[...], _NEG)
    q = q + jnp.where(m1_ref[...] > 0.0, s1_ref[...], _NEG)
    q = q + jnp.where(mL_ref[...] > 0.0, sL_ref[...], _NEG)
    q = q + jnp.where(m2_ref[...] > 0.0, s2_ref[...], _NEG)
    out_ref[...] = q


def _tc_combine(sA, s1, sL, s2, mA, m1, mL, m2):
    spec = pl.BlockSpec(sA.shape, lambda: (0, 0))
    return pl.pallas_call(
        _combine_body,
        in_specs=[spec] * 8,
        out_specs=spec,
        out_shape=jax.ShapeDtypeStruct(sA.shape, jnp.float32),
    )(sA, s1, sL, s2, mA, m1, mL, m2)


def kernel(encoded, action_types, object1, location, object2,
           action_type_masks, object1_masks, location_masks, object2_masks,
           W_at, b_at, W_o1, b_o1, W_loc, b_loc, W_o2, b_o2):
    B, H = encoded.shape
    A = action_type_masks.shape[1]
    O = object1_masks.shape[2]
    L = location_masks.shape[2]

    at = action_types[:, 0].astype(jnp.int32)
    o1i = object1[:, 0].astype(jnp.int32)
    loci = location[:, 0].astype(jnp.int32)
    o2i = object2[:, 0].astype(jnp.int32)

    tA = action_type_masks.transpose(1, 0)
    t1 = object1_masks.transpose(1, 2, 0).reshape(A * O, B)
    tL = location_masks.transpose(1, 2, 0).reshape(A * L, B)
    t2 = object2_masks.transpose(1, 2, 3, 0).reshape(A * O * O, B)

    mA, m1, mL, m2 = _sc_gather_masks(
        tA, t1, tL, t2, jnp.stack([at, o1i, loci, o2i]), O, L)

    i3 = (B // _BLK, _BLK // 128, 128)
    sA, s1, sL, s2 = _tc_scores(
        encoded,
        W_at.transpose(1, 0), W_o1.transpose(1, 0), W_loc.transpose(1, 0),
        W_o2.transpose(1, 0),
        at.reshape(i3), o1i.reshape(i3), loci.reshape(i3), o2i.reshape(i3),
        b_at.reshape(1, A), b_o1.reshape(1, O), b_loc.reshape(1, L),
        b_o2.reshape(1, O))

    t8 = (B // 128, 128)
    q = _tc_combine(sA.reshape(t8), s1.reshape(t8), sL.reshape(t8),
                    s2.reshape(t8), mA.reshape(t8), m1.reshape(t8),
                    mL.reshape(t8), m2.reshape(t8))
    return q.reshape(B, 1)


# speedup vs baseline: 1.2429x; 1.0258x over previous
import functools

import jax
import jax.numpy as jnp
import numpy as np
from jax import lax
from jax.experimental import pallas as pl
from jax.experimental.pallas import tpu as pltpu
from jax.experimental.pallas import tpu_sc as plsc

_NEG = np.float32(-1e9)
_BLK = 512
_NT = (((1,), (1,)), ((), ()))


def _sc_gather_masks(tA, t1, tL, t2, ati, o1i, loci, o2i, O, L):
    B = ati.shape[0]
    info = plsc.get_sparse_core_info()
    nw = info.num_cores * info.num_subcores
    bw = B // nw
    mesh = plsc.VectorSubcoreMesh(core_axis_name="c", subcore_axis_name="s")
    f32 = jnp.float32

    @functools.partial(
        pl.kernel,
        mesh=mesh,
        out_type=tuple(jax.ShapeDtypeStruct((B,), f32) for _ in range(4)),
        scratch_types=[
            pltpu.VMEM((4, bw), jnp.int32),
            pltpu.VMEM((4, bw), jnp.int32),
            pltpu.VMEM((4, bw), f32),
            pltpu.SemaphoreType.DMA,
            pltpu.SemaphoreType.DMA,
            pltpu.SemaphoreType.DMA,
            pltpu.SemaphoreType.DMA,
            pltpu.SemaphoreType.DMA,
        ],
    )
    def k(tA_h, t1_h, tL_h, t2_h, at_h, o1_h, loc_h, o2_h,
          oA_h, o1m_h, oL_h, o2m_h, iv, idx_v, outb,
          s0, s1, s2, s3, so):
        wid = lax.axis_index("s") * info.num_cores + lax.axis_index("c")
        base = wid * bw
        cb = (base // 128) * 128
        co = base - cb
        iloads = [
            pltpu.async_copy(h.at[pl.ds(base, bw)], iv.at[i], sem)
            for i, (h, sem) in enumerate(
                ((at_h, s0), (o1_h, s1), (loc_h, s2), (o2_h, s3)))
        ]
        for c in iloads:
            c.wait()
        for g in range(bw // 16):
            sl = pl.ds(g * 16, 16)
            atv = iv[0, sl]
            o1v = iv[1, sl]
            locv = iv[2, sl]
            o2v = iv[3, sl]
            idx_v[0, sl] = atv
            idx_v[1, sl] = atv * O + o1v
            idx_v[2, sl] = atv * L + locv
            idx_v[3, sl] = (atv * O + o1v) * O + o2v

        def body(bA_v, b1_v, bL_v, b2_v):
            tabs = ((tA_h, bA_v, s0), (t1_h, b1_v, s1),
                    (tL_h, bL_v, s2), (t2_h, b2_v, s3))
            copies = [
                pltpu.async_copy(t_h.at[idx_v.at[i], pl.ds(cb, 128)], b_v,
                                 sem)
                for i, (t_h, b_v, sem) in enumerate(tabs)
            ]
            for c in copies:
                c.wait()
            it = lax.iota(jnp.int32, 16)
            for cv in range(0, 128, bw):
                @pl.when(co == cv)
                def _extract(cv=cv):
                    for i, (_, b_v, _) in enumerate(tabs):
                        for g in range(bw // 16):
                            acc = jnp.zeros((16,), f32)
                            for r in range(16):
                                row = b_v[g * 16 + r,
                                          pl.ds(cv + g * 16, 16)]
                                acc = jnp.where(it == r, row, acc)
                            outb[i, pl.ds(g * 16, 16)] = acc
            outs = [
                pltpu.async_copy(outb.at[i], o_h.at[pl.ds(base, bw)], so)
                for i, o_h in enumerate((oA_h, o1m_h, oL_h, o2m_h))
            ]
            for o in outs:
                o.wait()

        pl.run_scoped(body, *[pltpu.VMEM((bw, 128), f32)] * 4)

    return k(tA, t1, tL, t2, ati, o1i, loci, o2i)


def _scores_body(enc_ref, watt_ref, w1t_ref, wlt_ref, w2t_ref,
                 at_ref, o1_ref, loc_ref, o2_ref,
                 bat_ref, bo1_ref, bloc_ref, bo2_ref,
                 oA_ref, o1_ref_out, oL_ref, o2_ref_out):
    f32 = jnp.float32
    H = enc_ref.shape[1]
    A = watt_ref.shape[0]
    O = w1t_ref.shape[0]
    n = enc_ref.shape[0]
    enc = enc_ref[...]

    def nt(x, y):
        return lax.dot_general(x, y, _NT, preferred_element_type=f32)

    def to_col(ref):
        t = jnp.swapaxes(ref[0], 0, 1)
        return jnp.concatenate(
            [t[:, i:i + 1] for i in range(t.shape[1])], axis=0)

    at = to_col(at_ref)
    o1 = to_col(o1_ref)
    loc = to_col(loc_ref)
    o2 = to_col(o2_ref)
    ioA = lax.broadcasted_iota(jnp.int32, (n, A), 1)
    io64 = lax.broadcasted_iota(jnp.int32, (n, O), 1)
    oh_at = (ioA == at).astype(f32)
    oh_o1 = (io64 == o1).astype(f32)
    oh_loc = (io64 == loc).astype(f32)
    oh_o2 = (io64 == o2).astype(f32)

    w1t = w1t_ref[...]
    wlt = wlt_ref[...]
    w2t = w2t_ref[...]

    row_at = nt(enc, watt_ref[...]) + bat_ref[...]
    oA_ref[...] = jnp.sum(oh_at * row_at, axis=1)

    row1 = nt(enc, w1t[:, :H]) + nt(oh_at, w1t[:, H:]) + bo1_ref[...]
    o1_ref_out[...] = jnp.sum(oh_o1 * row1, axis=1)

    rowL = (nt(enc, wlt[:, :H]) + nt(oh_at, wlt[:, H:H + A])
            + nt(oh_o1, wlt[:, H + A:]) + bloc_ref[...])
    oL_ref[...] = jnp.sum(oh_loc * rowL, axis=1)

    row2 = (nt(enc, w2t[:, :H]) + nt(oh_at, w2t[:, H:H + A])
            + nt(oh_o1, w2t[:, H + A:H + A + O])
            + nt(oh_loc, w2t[:, H + A + O:]) + bo2_ref[...])
    o2_ref_out[...] = jnp.sum(oh_o2 * row2, axis=1)


def _tc_scores(enc, watt, w1t, wlt, w2t, at, o1, loc, o2,
               b_at, b_o1, b_loc, b_o2):
    B, H = enc.shape

    def rows(i):
        return (i, 0)

    def full(i):
        return (0, 0)

    def fixed(a):
        return pl.BlockSpec(a.shape, full)

    def mrow(a):
        return pl.BlockSpec((1, _BLK // 128, 128), lambda i: (i, 0, 0))

    ospec = pl.BlockSpec((_BLK,), lambda i: (i,))
    oshape = jax.ShapeDtypeStruct((B,), jnp.float32)
    return pl.pallas_call(
        _scores_body,
        grid=(B // _BLK,),
        in_specs=[
            pl.BlockSpec((_BLK, H), rows),
            fixed(watt), fixed(w1t), fixed(wlt), fixed(w2t),
            mrow(at), mrow(o1), mrow(loc), mrow(o2),
            fixed(b_at), fixed(b_o1), fixed(b_loc), fixed(b_o2),
        ],
        out_specs=(ospec, ospec, ospec, ospec),
        out_shape=(oshape, oshape, oshape, oshape),
    )(enc, watt, w1t, wlt, w2t, at, o1, loc, o2, b_at, b_o1, b_loc, b_o2)


def _combine_body(sA_ref, s1_ref, sL_ref, s2_ref,
                  mA_ref, m1_ref, mL_ref, m2_ref, out_ref):
    q = jnp.where(mA_ref[...] > 0.0, sA_ref[...], _NEG)
    q = q + jnp.where(m1_ref[...] > 0.0, s1_ref[...], _NEG)
    q = q + jnp.where(mL_ref[...] > 0.0, sL_ref[...], _NEG)
    q = q + jnp.where(m2_ref[...] > 0.0, s2_ref[...], _NEG)
    out_ref[...] = q


def _tc_combine(sA, s1, sL, s2, mA, m1, mL, m2):
    spec = pl.BlockSpec(sA.shape, lambda: (0, 0))
    return pl.pallas_call(
        _combine_body,
        in_specs=[spec] * 8,
        out_specs=spec,
        out_shape=jax.ShapeDtypeStruct(sA.shape, jnp.float32),
    )(sA, s1, sL, s2, mA, m1, mL, m2)


def kernel(encoded, action_types, object1, location, object2,
           action_type_masks, object1_masks, location_masks, object2_masks,
           W_at, b_at, W_o1, b_o1, W_loc, b_loc, W_o2, b_o2):
    B, H = encoded.shape
    A = action_type_masks.shape[1]
    O = object1_masks.shape[2]
    L = location_masks.shape[2]

    at = action_types[:, 0].astype(jnp.int32)
    o1i = object1[:, 0].astype(jnp.int32)
    loci = location[:, 0].astype(jnp.int32)
    o2i = object2[:, 0].astype(jnp.int32)

    tA = action_type_masks.transpose(1, 0)
    t1 = object1_masks.transpose(1, 2, 0).reshape(A * O, B)
    tL = location_masks.transpose(1, 2, 0).reshape(A * L, B)
    t2 = object2_masks.transpose(1, 2, 3, 0).reshape(A * O * O, B)

    mA, m1, mL, m2 = _sc_gather_masks(tA, t1, tL, t2, at, o1i, loci, o2i,
                                      O, L)

    i3 = (B // _BLK, _BLK // 128, 128)
    sA, s1, sL, s2 = _tc_scores(
        encoded,
        W_at.transpose(1, 0), W_o1.transpose(1, 0), W_loc.transpose(1, 0),
        W_o2.transpose(1, 0),
        at.reshape(i3), o1i.reshape(i3), loci.reshape(i3), o2i.reshape(i3),
        b_at.reshape(1, A), b_o1.reshape(1, O), b_loc.reshape(1, L),
        b_o2.reshape(1, O))

    t8 = (B // 128, 128)
    q = _tc_combine(sA.reshape(t8), s1.reshape(t8), sL.reshape(t8),
                    s2.reshape(t8), mA.reshape(t8), m1.reshape(t8),
                    mL.reshape(t8), m2.reshape(t8))
    return q.reshape(B, 1)


# speedup vs baseline: 1.2451x; 1.0018x over previous
import functools

import jax
import jax.numpy as jnp
import numpy as np
from jax import lax
from jax.experimental import pallas as pl
from jax.experimental.pallas import tpu as pltpu
from jax.experimental.pallas import tpu_sc as plsc

_NEG = np.float32(-1e9)
_BLK = 512
_NT = (((1,), (1,)), ((), ()))


def _sc_gather_masks(tA, t1, tL, t2, ati, o1i, loci, o2i, O, L):
    B = ati.shape[0]
    info = plsc.get_sparse_core_info()
    nw = info.num_cores * info.num_subcores
    bw = B // nw
    mesh = plsc.VectorSubcoreMesh(core_axis_name="c", subcore_axis_name="s")
    f32 = jnp.float32

    @functools.partial(
        pl.kernel,
        mesh=mesh,
        out_type=tuple(jax.ShapeDtypeStruct((B,), f32) for _ in range(4)),
        scratch_types=[
            pltpu.VMEM((4, bw), jnp.int32),
            pltpu.VMEM((4, bw), jnp.int32),
            pltpu.VMEM((4, bw), f32),
            pltpu.SemaphoreType.DMA,
            pltpu.SemaphoreType.DMA,
            pltpu.SemaphoreType.DMA,
            pltpu.SemaphoreType.DMA,
            pltpu.SemaphoreType.DMA,
        ],
    )
    def k(tA_h, t1_h, tL_h, t2_h, at_h, o1_h, loc_h, o2_h,
          oA_h, o1m_h, oL_h, o2m_h, iv, idx_v, outb,
          s0, s1, s2, s3, so):
        wid = lax.axis_index("s") * info.num_cores + lax.axis_index("c")
        base = wid * bw
        cb = (base // 128) * 128
        co = base - cb
        iloads = [
            pltpu.async_copy(h.at[pl.ds(base, bw)], iv.at[i], sem)
            for i, (h, sem) in enumerate(
                ((at_h, s0), (o1_h, s1), (loc_h, s2), (o2_h, s3)))
        ]
        for c in iloads:
            c.wait()
        for g in range(bw // 16):
            sl = pl.ds(g * 16, 16)
            atv = iv[0, sl]
            o1v = iv[1, sl]
            locv = iv[2, sl]
            o2v = iv[3, sl]
            idx_v[0, sl] = atv
            idx_v[1, sl] = atv * O + o1v
            idx_v[2, sl] = atv * L + locv
            idx_v[3, sl] = (atv * O + o1v) * O + o2v

        def body(bA_v, b1_v, bL_v, b2_v):
            tabs = ((tA_h, bA_v, s0), (t1_h, b1_v, s1),
                    (tL_h, bL_v, s2), (t2_h, b2_v, s3))
            copies = [
                pltpu.async_copy(t_h.at[idx_v.at[i], pl.ds(cb, 128)], b_v,
                                 sem)
                for i, (t_h, b_v, sem) in enumerate(tabs)
            ]
            for c in copies:
                c.wait()
            it = lax.iota(jnp.int32, 16)
            for cv in range(0, 128, bw):
                @pl.when(co == cv)
                def _extract(cv=cv):
                    for i, (_, b_v, _) in enumerate(tabs):
                        for g in range(bw // 16):
                            acc = jnp.zeros((16,), f32)
                            for r in range(16):
                                row = b_v[g * 16 + r,
                                          pl.ds(cv + g * 16, 16)]
                                acc = jnp.where(it == r, row, acc)
                            outb[i, pl.ds(g * 16, 16)] = acc
            outs = [
                pltpu.async_copy(outb.at[i], o_h.at[pl.ds(base, bw)], so)
                for i, o_h in enumerate((oA_h, o1m_h, oL_h, o2m_h))
            ]
            for o in outs:
                o.wait()

        pl.run_scoped(body, *[pltpu.VMEM((bw, 128), f32)] * 4)

    return k(tA, t1, tL, t2, ati, o1i, loci, o2i)


def _scores_body(enc_ref, watt_ref, w1t_ref, wlt_ref, w2t_ref,
                 at_ref, o1_ref, loc_ref, o2_ref,
                 bat_ref, bo1_ref, bloc_ref, bo2_ref,
                 oA_ref, o1_ref_out, oL_ref, o2_ref_out):
    f32 = jnp.float32
    H = enc_ref.shape[1]
    A = watt_ref.shape[0]
    O = w1t_ref.shape[0]
    n = enc_ref.shape[0]
    enc = enc_ref[...]

    def nt(x, y):
        return lax.dot_general(x, y, _NT, preferred_element_type=f32)

    def to_col(ref):
        t = jnp.swapaxes(ref[0], 0, 1)
        return jnp.concatenate(
            [t[:, i:i + 1] for i in range(t.shape[1])], axis=0)

    at = to_col(at_ref)
    o1 = to_col(o1_ref)
    loc = to_col(loc_ref)
    o2 = to_col(o2_ref)
    ioA = lax.broadcasted_iota(jnp.int32, (n, A), 1)
    io64 = lax.broadcasted_iota(jnp.int32, (n, O), 1)
    oh_at = (ioA == at).astype(f32)
    oh_o1 = (io64 == o1).astype(f32)
    oh_loc = (io64 == loc).astype(f32)
    oh_o2 = (io64 == o2).astype(f32)

    w1t = w1t_ref[...]
    wlt = wlt_ref[...]
    w2t = w2t_ref[...]

    onesA = jnp.full((1, A), 1.0, f32)
    onesO = jnp.full((1, O), 1.0, f32)

    row_at = nt(enc, watt_ref[...]) + bat_ref[...]
    cA = nt(oh_at * row_at, onesA)

    row1 = nt(enc, w1t[:, :H]) + nt(oh_at, w1t[:, H:]) + bo1_ref[...]
    c1 = nt(oh_o1 * row1, onesO)

    rowL = (nt(enc, wlt[:, :H]) + nt(oh_at, wlt[:, H:H + A])
            + nt(oh_o1, wlt[:, H + A:]) + bloc_ref[...])
    cL = nt(oh_loc * rowL, onesO)

    row2 = (nt(enc, w2t[:, :H]) + nt(oh_at, w2t[:, H:H + A])
            + nt(oh_o1, w2t[:, H + A:H + A + O])
            + nt(oh_loc, w2t[:, H + A + O:]) + bo2_ref[...])
    c2 = nt(oh_o2 * row2, onesO)

    T = jnp.swapaxes(jnp.concatenate([cA, c1, cL, c2], axis=1), 0, 1)
    oA_ref[...] = T[0, :]
    o1_ref_out[...] = T[1, :]
    oL_ref[...] = T[2, :]
    o2_ref_out[...] = T[3, :]


def _tc_scores(enc, watt, w1t, wlt, w2t, at, o1, loc, o2,
               b_at, b_o1, b_loc, b_o2):
    B, H = enc.shape

    def rows(i):
        return (i, 0)

    def full(i):
        return (0, 0)

    def fixed(a):
        return pl.BlockSpec(a.shape, full)

    def mrow(a):
        return pl.BlockSpec((1, _BLK // 128, 128), lambda i: (i, 0, 0))

    ospec = pl.BlockSpec((_BLK,), lambda i: (i,))
    oshape = jax.ShapeDtypeStruct((B,), jnp.float32)
    return pl.pallas_call(
        _scores_body,
        grid=(B // _BLK,),
        in_specs=[
            pl.BlockSpec((_BLK, H), rows),
            fixed(watt), fixed(w1t), fixed(wlt), fixed(w2t),
            mrow(at), mrow(o1), mrow(loc), mrow(o2),
            fixed(b_at), fixed(b_o1), fixed(b_loc), fixed(b_o2),
        ],
        out_specs=(ospec, ospec, ospec, ospec),
        out_shape=(oshape, oshape, oshape, oshape),
    )(enc, watt, w1t, wlt, w2t, at, o1, loc, o2, b_at, b_o1, b_loc, b_o2)


def _combine_body(sA_ref, s1_ref, sL_ref, s2_ref,
                  mA_ref, m1_ref, mL_ref, m2_ref, out_ref):
    q = jnp.where(mA_ref[...] > 0.0, sA_ref[...], _NEG)
    q = q + jnp.where(m1_ref[...] > 0.0, s1_ref[...], _NEG)
    q = q + jnp.where(mL_ref[...] > 0.0, sL_ref[...], _NEG)
    q = q + jnp.where(m2_ref[...] > 0.0, s2_ref[...], _NEG)
    out_ref[...] = q


def _tc_combine(sA, s1, sL, s2, mA, m1, mL, m2):
    spec = pl.BlockSpec(sA.shape, lambda: (0, 0))
    return pl.pallas_call(
        _combine_body,
        in_specs=[spec] * 8,
        out_specs=spec,
        out_shape=jax.ShapeDtypeStruct(sA.shape, jnp.float32),
    )(sA, s1, sL, s2, mA, m1, mL, m2)


def kernel(encoded, action_types, object1, location, object2,
           action_type_masks, object1_masks, location_masks, object2_masks,
           W_at, b_at, W_o1, b_o1, W_loc, b_loc, W_o2, b_o2):
    B, H = encoded.shape
    A = action_type_masks.shape[1]
    O = object1_masks.shape[2]
    L = location_masks.shape[2]

    at = action_types[:, 0].astype(jnp.int32)
    o1i = object1[:, 0].astype(jnp.int32)
    loci = location[:, 0].astype(jnp.int32)
    o2i = object2[:, 0].astype(jnp.int32)

    tA = action_type_masks.transpose(1, 0)
    t1 = object1_masks.transpose(1, 2, 0).reshape(A * O, B)
    tL = location_masks.transpose(1, 2, 0).reshape(A * L, B)
    t2 = object2_masks.transpose(1, 2, 3, 0).reshape(A * O * O, B)

    mA, m1, mL, m2 = _sc_gather_masks(tA, t1, tL, t2, at, o1i, loci, o2i,
                                      O, L)

    i3 = (B // _BLK, _BLK // 128, 128)
    sA, s1, sL, s2 = _tc_scores(
        encoded,
        W_at.transpose(1, 0), W_o1.transpose(1, 0), W_loc.transpose(1, 0),
        W_o2.transpose(1, 0),
        at.reshape(i3), o1i.reshape(i3), loci.reshape(i3), o2i.reshape(i3),
        b_at.reshape(1, A), b_o1.reshape(1, O), b_loc.reshape(1, L),
        b_o2.reshape(1, O))

    t8 = (B // 128, 128)
    q = _tc_combine(sA.reshape(t8), s1.reshape(t8), sL.reshape(t8),
                    s2.reshape(t8), mA.reshape(t8), m1.reshape(t8),
                    mL.reshape(t8), m2.reshape(t8))
    return q.reshape(B, 1)


# speedup vs baseline: 1.2638x; 1.0150x over previous
import functools

import jax
import jax.numpy as jnp
import numpy as np
from jax import lax
from jax.experimental import pallas as pl
from jax.experimental.pallas import tpu as pltpu
from jax.experimental.pallas import tpu_sc as plsc

_NEG = np.float32(-1e9)
_BLK = 256
_NT = (((1,), (1,)), ((), ()))


def _sc_gather_masks(tA, t1, tL, t2, ati, o1i, loci, o2i, O, L):
    B = ati.shape[0]
    info = plsc.get_sparse_core_info()
    nw = info.num_cores * info.num_subcores
    bw = B // nw
    mesh = plsc.VectorSubcoreMesh(core_axis_name="c", subcore_axis_name="s")
    f32 = jnp.float32

    @functools.partial(
        pl.kernel,
        mesh=mesh,
        out_type=tuple(jax.ShapeDtypeStruct((B,), f32) for _ in range(4)),
        scratch_types=[
            pltpu.VMEM((4, bw), jnp.int32),
            pltpu.VMEM((4, bw), jnp.int32),
            pltpu.VMEM((4, bw), f32),
            pltpu.SemaphoreType.DMA,
            pltpu.SemaphoreType.DMA,
            pltpu.SemaphoreType.DMA,
            pltpu.SemaphoreType.DMA,
            pltpu.SemaphoreType.DMA,
        ],
    )
    def k(tA_h, t1_h, tL_h, t2_h, at_h, o1_h, loc_h, o2_h,
          oA_h, o1m_h, oL_h, o2m_h, iv, idx_v, outb,
          s0, s1, s2, s3, so):
        wid = lax.axis_index("s") * info.num_cores + lax.axis_index("c")
        base = wid * bw
        cb = (base // 128) * 128
        co = base - cb
        iloads = [
            pltpu.async_copy(h.at[pl.ds(base, bw)], iv.at[i], sem)
            for i, (h, sem) in enumerate(
                ((at_h, s0), (o1_h, s1), (loc_h, s2), (o2_h, s3)))
        ]
        for c in iloads:
            c.wait()
        for g in range(bw // 16):
            sl = pl.ds(g * 16, 16)
            atv = iv[0, sl]
            o1v = iv[1, sl]
            locv = iv[2, sl]
            o2v = iv[3, sl]
            idx_v[0, sl] = atv
            idx_v[1, sl] = atv * O + o1v
            idx_v[2, sl] = atv * L + locv
            idx_v[3, sl] = (atv * O + o1v) * O + o2v

        def body(bA_v, b1_v, bL_v, b2_v):
            tabs = ((tA_h, bA_v, s0), (t1_h, b1_v, s1),
                    (tL_h, bL_v, s2), (t2_h, b2_v, s3))
            copies = [
                pltpu.async_copy(t_h.at[idx_v.at[i], pl.ds(cb, 128)], b_v,
                                 sem)
                for i, (t_h, b_v, sem) in enumerate(tabs)
            ]
            for c in copies:
                c.wait()
            it = lax.iota(jnp.int32, 16)
            for cv in range(0, 128, bw):
                @pl.when(co == cv)
                def _extract(cv=cv):
                    for i, (_, b_v, _) in enumerate(tabs):
                        for g in range(bw // 16):
                            acc = jnp.zeros((16,), f32)
                            for r in range(16):
                                row = b_v[g * 16 + r,
                                          pl.ds(cv + g * 16, 16)]
                                acc = jnp.where(it == r, row, acc)
                            outb[i, pl.ds(g * 16, 16)] = acc
            outs = [
                pltpu.async_copy(outb.at[i], o_h.at[pl.ds(base, bw)], so)
                for i, o_h in enumerate((oA_h, o1m_h, oL_h, o2m_h))
            ]
            for o in outs:
                o.wait()

        pl.run_scoped(body, *[pltpu.VMEM((bw, 128), f32)] * 4)

    return k(tA, t1, tL, t2, ati, o1i, loci, o2i)


def _scores_body(enc_ref, watt_ref, w1t_ref, wlt_ref, w2t_ref,
                 at_ref, o1_ref, loc_ref, o2_ref,
                 bat_ref, bo1_ref, bloc_ref, bo2_ref,
                 oA_ref, o1_ref_out, oL_ref, o2_ref_out):
    f32 = jnp.float32
    H = enc_ref.shape[1]
    A = watt_ref.shape[0]
    O = w1t_ref.shape[0]
    n = enc_ref.shape[0]
    enc = enc_ref[...]

    def nt(x, y):
        return lax.dot_general(x, y, _NT, preferred_element_type=f32)

    def to_col(ref):
        t = jnp.swapaxes(ref[0], 0, 1)
        return jnp.concatenate(
            [t[:, i:i + 1] for i in range(t.shape[1])], axis=0)

    at = to_col(at_ref)
    o1 = to_col(o1_ref)
    loc = to_col(loc_ref)
    o2 = to_col(o2_ref)
    ioA = lax.broadcasted_iota(jnp.int32, (n, A), 1)
    io64 = lax.broadcasted_iota(jnp.int32, (n, O), 1)
    oh_at = (ioA == at).astype(f32)
    oh_o1 = (io64 == o1).astype(f32)
    oh_loc = (io64 == loc).astype(f32)
    oh_o2 = (io64 == o2).astype(f32)

    w1t = w1t_ref[...]
    wlt = wlt_ref[...]
    w2t = w2t_ref[...]

    onesA = jnp.full((1, A), 1.0, f32)
    onesO = jnp.full((1, O), 1.0, f32)

    row_at = nt(enc, watt_ref[...]) + bat_ref[...]
    cA = nt(oh_at * row_at, onesA)

    row1 = nt(enc, w1t[:, :H]) + nt(oh_at, w1t[:, H:]) + bo1_ref[...]
    c1 = nt(oh_o1 * row1, onesO)

    rowL = (nt(enc, wlt[:, :H]) + nt(oh_at, wlt[:, H:H + A])
            + nt(oh_o1, wlt[:, H + A:]) + bloc_ref[...])
    cL = nt(oh_loc * rowL, onesO)

    row2 = (nt(enc, w2t[:, :H]) + nt(oh_at, w2t[:, H:H + A])
            + nt(oh_o1, w2t[:, H + A:H + A + O])
            + nt(oh_loc, w2t[:, H + A + O:]) + bo2_ref[...])
    c2 = nt(oh_o2 * row2, onesO)

    T = jnp.swapaxes(jnp.concatenate([cA, c1, cL, c2], axis=1), 0, 1)
    oA_ref[...] = T[0, :]
    o1_ref_out[...] = T[1, :]
    oL_ref[...] = T[2, :]
    o2_ref_out[...] = T[3, :]


def _tc_scores(enc, watt, w1t, wlt, w2t, at, o1, loc, o2,
               b_at, b_o1, b_loc, b_o2):
    B, H = enc.shape

    def rows(i):
        return (i, 0)

    def full(i):
        return (0, 0)

    def fixed(a):
        return pl.BlockSpec(a.shape, full)

    def mrow(a):
        return pl.BlockSpec((1, _BLK // 128, 128), lambda i: (i, 0, 0))

    ospec = pl.BlockSpec((_BLK,), lambda i: (i,))
    oshape = jax.ShapeDtypeStruct((B,), jnp.float32)
    return pl.pallas_call(
        _scores_body,
        grid=(B // _BLK,),
        in_specs=[
            pl.BlockSpec((_BLK, H), rows),
            fixed(watt), fixed(w1t), fixed(wlt), fixed(w2t),
            mrow(at), mrow(o1), mrow(loc), mrow(o2),
            fixed(b_at), fixed(b_o1), fixed(b_loc), fixed(b_o2),
        ],
        out_specs=(ospec, ospec, ospec, ospec),
        out_shape=(oshape, oshape, oshape, oshape),
    )(enc, watt, w1t, wlt, w2t, at, o1, loc, o2, b_at, b_o1, b_loc, b_o2)


def _combine_body(sA_ref, s1_ref, sL_ref, s2_ref,
                  mA_ref, m1_ref, mL_ref, m2_ref, out_ref):
    q = jnp.where(mA_ref[...] > 0.0, sA_ref[...], _NEG)
    q = q + jnp.where(m1_ref[...] > 0.0, s1_ref[...], _NEG)
    q = q + jnp.where(mL_ref[...] > 0.0, sL_ref[...], _NEG)
    q = q + jnp.where(m2_ref[...] > 0.0, s2_ref[...], _NEG)
    out_ref[...] = q


def _tc_combine(sA, s1, sL, s2, mA, m1, mL, m2):
    spec = pl.BlockSpec(sA.shape, lambda: (0, 0))
    return pl.pallas_call(
        _combine_body,
        in_specs=[spec] * 8,
        out_specs=spec,
        out_shape=jax.ShapeDtypeStruct(sA.shape, jnp.float32),
    )(sA, s1, sL, s2, mA, m1, mL, m2)


def kernel(encoded, action_types, object1, location, object2,
           action_type_masks, object1_masks, location_masks, object2_masks,
           W_at, b_at, W_o1, b_o1, W_loc, b_loc, W_o2, b_o2):
    B, H = encoded.shape
    A = action_type_masks.shape[1]
    O = object1_masks.shape[2]
    L = location_masks.shape[2]

    at = action_types[:, 0].astype(jnp.int32)
    o1i = object1[:, 0].astype(jnp.int32)
    loci = location[:, 0].astype(jnp.int32)
    o2i = object2[:, 0].astype(jnp.int32)

    tA = action_type_masks.transpose(1, 0)
    t1 = object1_masks.transpose(1, 2, 0).reshape(A * O, B)
    tL = location_masks.transpose(1, 2, 0).reshape(A * L, B)
    t2 = object2_masks.transpose(1, 2, 3, 0).reshape(A * O * O, B)

    mA, m1, mL, m2 = _sc_gather_masks(tA, t1, tL, t2, at, o1i, loci, o2i,
                                      O, L)

    i3 = (B // _BLK, _BLK // 128, 128)
    sA, s1, sL, s2 = _tc_scores(
        encoded,
        W_at.transpose(1, 0), W_o1.transpose(1, 0), W_loc.transpose(1, 0),
        W_o2.transpose(1, 0),
        at.reshape(i3), o1i.reshape(i3), loci.reshape(i3), o2i.reshape(i3),
        b_at.reshape(1, A), b_o1.reshape(1, O), b_loc.reshape(1, L),
        b_o2.reshape(1, O))

    t8 = (B // 128, 128)
    q = _tc_combine(sA.reshape(t8), s1.reshape(t8), sL.reshape(t8),
                    s2.reshape(t8), mA.reshape(t8), m1.reshape(t8),
                    mL.reshape(t8), m2.reshape(t8))
    return q.reshape(B, 1)
